# Initial kernel scaffold; baseline (speedup 1.0000x reference)
import jax
import jax.numpy as jnp
from jax.experimental import pallas as pl


def kernel(X, seg, word_embd, segment_embd, position_embd, gamma, beta):
    raise NotImplementedError("write your pallas kernel here")


# speedup vs baseline: 5.1494x; 5.1494x over previous
import functools

import jax
import jax.numpy as jnp
from jax import lax
from jax.experimental import pallas as pl
from jax.experimental.pallas import tpu as pltpu
from jax.experimental.pallas import tpu_sc as plsc

LANES = 16
NC, NS = 2, 16
NW = NC * NS
CH = 128


def _rsqrt_newton(a):
    ai = plsc.bitcast(a, jnp.int32)
    yi = jnp.int32(0x5F3759DF) - lax.shift_right_logical(ai, 1)
    y = plsc.bitcast(yi, jnp.float32)
    half = a * 0.5
    for _ in range(3):
        y = y * (1.5 - half * y * y)
    return y


def _make_kernel(N, V, E, L_seq, NSEG):
    per_w = N // NW
    n_ch = per_w // CH
    ne = E // LANES

    mesh = plsc.VectorSubcoreMesh(core_axis_name="c", subcore_axis_name="s")

    @functools.partial(
        pl.kernel,
        out_type=jax.ShapeDtypeStruct((N, E), jnp.float32),
        mesh=mesh,
        compiler_params=pltpu.CompilerParams(needs_layout_passes=False),
        scratch_types=[
            pltpu.VMEM((L_seq, E), jnp.float32),
            pltpu.VMEM((NSEG, E), jnp.float32),
            pltpu.VMEM((E,), jnp.float32),
            pltpu.VMEM((E,), jnp.float32),
            pltpu.VMEM((CH + LANES,), jnp.int32),
            pltpu.VMEM((CH,), jnp.int32),
            pltpu.VMEM((CH + LANES,), jnp.int32),
            pltpu.VMEM((CH, 2 * E), jnp.float32),
            pltpu.VMEM((CH, E), jnp.float32),
            pltpu.SemaphoreType.DMA,
        ],
    )
    def body(x_hbm, s_hbm, wtab, stab, ptab, g_hbm, b_hbm, out_hbm,
             pos_v, seg_v, g_v, b_v, xi_v, xh_v, si_v, rows_v, out_v, sem):
        wid = lax.axis_index("s") * NC + lax.axis_index("c")
        w_base = wid * per_w

        pltpu.sync_copy(ptab.at[pl.ds(0, L_seq)], pos_v)
        pltpu.sync_copy(stab, seg_v)
        pltpu.sync_copy(g_hbm, g_v)
        pltpu.sync_copy(b_hbm, b_v)

        sr0 = [seg_v[0, pl.ds(e * LANES, LANES)] for e in range(ne)]
        sr1 = [seg_v[1, pl.ds(e * LANES, LANES)] for e in range(ne)]
        gv = [g_v[pl.ds(e * LANES, LANES)] for e in range(ne)]
        bv = [b_v[pl.ds(e * LANES, LANES)] for e in range(ne)]
        inv_e = jnp.float32(1.0 / E)

        def chunk(c, _):
            base = w_base + c * CH
            pltpu.sync_copy(x_hbm.at[pl.ds(base, CH)], xi_v.at[pl.ds(0, CH)])
            pltpu.sync_copy(s_hbm.at[pl.ds(base, CH)], si_v.at[pl.ds(0, CH)])
            for q in range(CH // LANES):
                sl = pl.ds(q * LANES, LANES)
                xh_v[sl] = lax.shift_right_logical(xi_v[sl], 1)
            pltpu.async_copy(wtab.at[xh_v], rows_v, sem).wait()

            @plsc.parallel_loop(0, CH, step=1, unroll=2)
            def _(j):
                g = base + j
                pi = lax.rem(g, L_seq)
                sv = si_v[pl.ds(j, LANES)][0]
                xv = xi_v[pl.ds(j, LANES)][0]
                off = (xv & 1) * E
                t = []
                for e in range(ne):
                    w = rows_v[j, pl.ds(off + e * LANES, LANES)]
                    p = pos_v[pi, pl.ds(e * LANES, LANES)]
                    s = seg_v[sv, pl.ds(e * LANES, LANES)]
                    t.append(w + p + s)
                tot = jnp.sum((t[0] + t[1]) + (t[2] + t[3]))
                sq = jnp.sum((t[0] * t[0] + t[1] * t[1])
                             + (t[2] * t[2] + t[3] * t[3]))
                mean = tot * inv_e
                var = sq * inv_e - mean * mean
                rv = _rsqrt_newton(jnp.full((LANES,), var + 1e-5))
                mv = jnp.full((LANES,), mean)
                for e in range(ne):
                    out_v[j, pl.ds(e * LANES, LANES)] = (
                        (t[e] - mv) * rv * gv[e] + bv[e])

            pltpu.sync_copy(out_v, out_hbm.at[pl.ds(base, CH)])
            return 0

        lax.fori_loop(0, n_ch, chunk, 0)

    return body


def kernel(X, seg, word_embd, segment_embd, position_embd, gamma, beta):
    B, L_seq = X.shape
    V, E = word_embd.shape
    NSEG = segment_embd.shape[0]
    N = B * L_seq
    xf = X.reshape(-1).astype(jnp.int32)
    sf = seg.reshape(-1).astype(jnp.int32)
    wtab2 = word_embd.reshape(V // 2, 2 * E)
    k = _make_kernel(N, V, E, L_seq, NSEG)
    out = k(xf, sf, wtab2, segment_embd, position_embd, gamma, beta)
    return out.reshape(B, L_seq, E)
